# baseline (speedup 1.0000x reference)
import functools

import jax
import jax.numpy as jnp
from jax import lax
from jax.experimental import pallas as pl
from jax.experimental.pallas import tpu as pltpu
from jax.experimental.pallas import tpu_sc as plsc

VOCAB = 1000
BATCH = 16384

_info = plsc.get_sparse_core_info()
NC, NS = _info.num_cores, _info.num_subcores
NW = NC * NS
B_PER_W = BATCH // NW
R = 64
CH = B_PER_W // R


def _gather_kernel(table, idx3):
    mesh = plsc.VectorSubcoreMesh(core_axis_name="c", subcore_axis_name="s")

    @functools.partial(
        pl.kernel,
        mesh=mesh,
        out_type=jax.ShapeDtypeStruct((BATCH, VOCAB), jnp.float32),
        scratch_types=[
            pltpu.VMEM((CH, R), jnp.int32),
            pltpu.VMEM((R, VOCAB), jnp.float32),
            pltpu.SemaphoreType.DMA,
        ],
        compiler_params=pltpu.CompilerParams(use_tc_tiling_on_sc=False),
    )
    def k(table_hbm, idx_hbm, out_hbm, idx_v, rows_v, sem):
        wid = lax.axis_index("s") * NC + lax.axis_index("c")
        base = wid * B_PER_W
        pltpu.sync_copy(idx_hbm.at[wid], idx_v)

        def chunk(c, carry):
            row0 = base + c * R
            pltpu.async_copy(table_hbm.at[idx_v.at[c]], rows_v, sem).wait()
            pltpu.sync_copy(rows_v, out_hbm.at[pl.ds(row0, R)])
            return carry

        lax.fori_loop(0, CH, chunk, 0)

    return k(table, idx3)


def kernel(idx, token_embedding_table):
    idx3 = idx.reshape(NW, CH, R)
    return _gather_kernel(token_embedding_table, idx3)


# speedup vs baseline: 1.0172x; 1.0172x over previous
import functools

import jax
import jax.numpy as jnp
from jax import lax
from jax.experimental import pallas as pl
from jax.experimental.pallas import tpu as pltpu
from jax.experimental.pallas import tpu_sc as plsc

VOCAB = 1000
BATCH = 16384

_info = plsc.get_sparse_core_info()
NC, NS = _info.num_cores, _info.num_subcores
NW = NC * NS
B_PER_W = BATCH // NW
R = 32
CH = B_PER_W // R


def _gather_kernel(table, idx3):
    mesh = plsc.VectorSubcoreMesh(core_axis_name="c", subcore_axis_name="s")

    @functools.partial(
        pl.kernel,
        mesh=mesh,
        out_type=jax.ShapeDtypeStruct((BATCH, VOCAB), jnp.float32),
        scratch_types=[
            pltpu.VMEM((CH, R), jnp.int32),
            pltpu.VMEM((R, VOCAB), jnp.float32),
            pltpu.VMEM((R, VOCAB), jnp.float32),
            pltpu.SemaphoreType.DMA,
            pltpu.SemaphoreType.DMA,
            pltpu.SemaphoreType.DMA,
            pltpu.SemaphoreType.DMA,
        ],
        compiler_params=pltpu.CompilerParams(use_tc_tiling_on_sc=False),
    )
    def k(table_hbm, idx_hbm, out_hbm, idx_v, buf_a, buf_b, ga, gb, wa, wb):
        wid = lax.axis_index("s") * NC + lax.axis_index("c")
        base = wid * B_PER_W
        pltpu.sync_copy(idx_hbm.at[wid], idx_v)

        bufs = (buf_a, buf_b)
        gsems = (ga, gb)
        wsems = (wa, wb)

        def gather(c, b):
            return pltpu.async_copy(
                table_hbm.at[idx_v.at[c]], bufs[b], gsems[b]
            )
        def write(c, b):
            return pltpu.async_copy(
                bufs[b], out_hbm.at[pl.ds(base + c * R, R)], wsems[b]
            )

        gh = [gather(0, 0), None]
        wh = [None, None]
        for c in range(CH):
            b = c % 2
            nb = 1 - b
            if c + 1 < CH:
                if wh[nb] is not None:
                    wh[nb].wait()
                gh[nb] = gather(c + 1, nb)
            gh[b].wait()
            wh[b] = write(c, b)
        wh[0].wait()
        wh[1].wait()

    return k(table, idx3)


def kernel(idx, token_embedding_table):
    idx3 = idx.reshape(NW, CH, R)
    return _gather_kernel(token_embedding_table, idx3)


# speedup vs baseline: 1.1028x; 1.0842x over previous
import functools

import jax
import jax.numpy as jnp
from jax import lax
from jax.experimental import pallas as pl
from jax.experimental.pallas import tpu as pltpu
from jax.experimental.pallas import tpu_sc as plsc

VOCAB = 1000
BATCH = 16384

_info = plsc.get_sparse_core_info()
NC, NS = _info.num_cores, _info.num_subcores
NW = NC * NS
B_PER_W = BATCH // NW
R = 32
CH = B_PER_W // R


def _gather_kernel(table, idx3):
    mesh = plsc.VectorSubcoreMesh(core_axis_name="c", subcore_axis_name="s")

    @functools.partial(
        pl.kernel,
        mesh=mesh,
        out_type=jax.ShapeDtypeStruct((BATCH, VOCAB), jnp.float32),
        scratch_types=[
            pltpu.VMEM_SHARED((VOCAB, VOCAB), jnp.float32),
            pltpu.VMEM((CH, R), jnp.int32),
            pltpu.VMEM((R, VOCAB), jnp.float32),
            pltpu.VMEM((R, VOCAB), jnp.float32),
            pltpu.SemaphoreType.DMA,
            pltpu.SemaphoreType.DMA,
            pltpu.SemaphoreType.DMA,
            pltpu.SemaphoreType.DMA,
        ],
        compiler_params=pltpu.CompilerParams(use_tc_tiling_on_sc=False),
    )
    def k(table_hbm, idx_hbm, out_hbm, table_sp, idx_v, buf_a, buf_b,
          ga, gb, wa, wb):
        sid = lax.axis_index("s")
        wid = sid * NC + lax.axis_index("c")
        base = wid * B_PER_W
        pltpu.sync_copy(idx_hbm.at[wid], idx_v)

        @pl.when(sid < 8)
        def _():
            lo = sid * 63
            pltpu.sync_copy(
                table_hbm.at[pl.ds(lo, 63)], table_sp.at[pl.ds(lo, 63)]
            )

        @pl.when(sid >= 8)
        def _():
            lo = 504 + (sid - 8) * 62
            pltpu.sync_copy(
                table_hbm.at[pl.ds(lo, 62)], table_sp.at[pl.ds(lo, 62)]
            )

        plsc.subcore_barrier()

        bufs = (buf_a, buf_b)
        gsems = (ga, gb)
        wsems = (wa, wb)

        def gather(c, b):
            return pltpu.async_copy(
                table_sp.at[idx_v.at[c]], bufs[b], gsems[b]
            )
        def write(c, b):
            return pltpu.async_copy(
                bufs[b], out_hbm.at[pl.ds(base + c * R, R)], wsems[b]
            )

        gh = [gather(0, 0), None]
        wh = [None, None]
        for c in range(CH):
            b = c % 2
            nb = 1 - b
            if c + 1 < CH:
                if wh[nb] is not None:
                    wh[nb].wait()
                gh[nb] = gather(c + 1, nb)
            gh[b].wait()
            wh[b] = write(c, b)
        wh[0].wait()
        wh[1].wait()

    return k(table, idx3)


def kernel(idx, token_embedding_table):
    idx3 = idx.reshape(NW, CH, R)
    return _gather_kernel(token_embedding_table, idx3)


# speedup vs baseline: 1.5754x; 1.4285x over previous
import functools

import jax
import jax.numpy as jnp
from jax import lax
from jax.experimental import pallas as pl
from jax.experimental.pallas import tpu as pltpu
from jax.experimental.pallas import tpu_sc as plsc

VOCAB = 1000
VPAD = 1024
BATCH = 16384

_info = plsc.get_sparse_core_info()
NC, NS = _info.num_cores, _info.num_subcores
NW = NC * NS
B_PER_W = BATCH // NW
R = 16
CH = B_PER_W // R


def _gather_kernel(table_pad, idx2):
    mesh = plsc.VectorSubcoreMesh(core_axis_name="c", subcore_axis_name="s")

    @functools.partial(
        pl.kernel,
        mesh=mesh,
        out_type=jax.ShapeDtypeStruct((BATCH, VPAD), jnp.float32),
        scratch_types=[
            pltpu.VMEM((CH, R), jnp.int32),
            pltpu.VMEM((R, VPAD), jnp.float32),
            pltpu.VMEM((R, VPAD), jnp.float32),
            pltpu.SemaphoreType.DMA,
            pltpu.SemaphoreType.DMA,
            pltpu.SemaphoreType.DMA,
            pltpu.SemaphoreType.DMA,
        ],
    )
    def k(table_hbm, idx_hbm, out_hbm, idx_v, buf_a, buf_b, ga, gb, wa, wb):
        sid = lax.axis_index("s")
        wid = sid * NC + lax.axis_index("c")
        base = wid * B_PER_W
        pltpu.sync_copy(idx_hbm.at[pl.ds(wid * CH, CH)], idx_v)

        bufs = (buf_a, buf_b)
        gsems = (ga, gb)
        wsems = (wa, wb)

        def gather(c, b):
            return pltpu.async_copy(
                table_hbm.at[idx_v.at[c]], bufs[b], gsems[b]
            )

        def write(c, b):
            return pltpu.async_copy(
                bufs[b], out_hbm.at[pl.ds(base + c * R, R)], wsems[b]
            )

        gh = [gather(0, 0), None]
        wh = [None, None]
        for c in range(CH):
            b = c % 2
            nb = 1 - b
            if c + 1 < CH:
                if wh[nb] is not None:
                    wh[nb].wait()
                gh[nb] = gather(c + 1, nb)
            gh[b].wait()
            wh[b] = write(c, b)
        wh[0].wait()
        wh[1].wait()

    return k(table_pad, idx2)


def kernel(idx, token_embedding_table):
    table_pad = jnp.pad(token_embedding_table, ((0, 0), (0, VPAD - VOCAB)))
    idx2 = idx.reshape(NW * CH, R)
    out_pad = _gather_kernel(table_pad, idx2)
    return out_pad[:, :VOCAB]
